# Initial kernel scaffold; baseline (speedup 1.0000x reference)
import jax
import jax.numpy as jnp
from jax.experimental import pallas as pl


def kernel(x, W_emb, b_emb, W_attn, b_attn, W_conv, b_conv, gamma, beta):
    raise NotImplementedError("write your pallas kernel here")


# speedup vs baseline: 29.1249x; 29.1249x over previous
import functools

import jax
import jax.numpy as jnp
from jax.experimental import pallas as pl
from jax.experimental.pallas import tpu as pltpu

KNN = 16


def _prep_body(xcn_ref, xnc_ref, wemb_ref, wattn_ref, beb_col_ref, beb_row_ref,
               battn_ref, xn_cn_ref, xn_nc_ref, sq_n_ref, sq_t_ref,
               s1_t_ref, s2_n_ref):
    C = xcn_ref.shape[1]
    xb_cn = xcn_ref[0]
    xb_nc = xnc_ref[0]
    nrm_col = jnp.sqrt(jnp.sum(xb_cn * xb_cn, axis=1, keepdims=True))
    inv_col = 1.0 / jnp.maximum(nrm_col, 1e-12)
    xn_cn = xb_cn * inv_col
    xn_cn_ref[0] = xn_cn
    nrm_row = jnp.sqrt(jnp.sum(xb_nc * xb_nc, axis=0, keepdims=True))
    inv_row = 1.0 / jnp.maximum(nrm_row, 1e-12)
    xn_nc = xb_nc * inv_row
    xn_nc_ref[0] = xn_nc
    sq_n_ref[0] = jnp.sum(xn_cn * xn_cn, axis=0, keepdims=True)
    sq_t_ref[0] = jnp.sum(xn_nc * xn_nc, axis=1, keepdims=True)
    wemb = wemb_ref[...]
    wattn = wattn_ref[...]
    a1 = wattn[:, :C]
    a2 = wattn[:, C:]
    e_nc = jax.lax.dot_general(xb_nc, wemb, (((1,), (1,)), ((), ())),
                               preferred_element_type=jnp.float32)
    e_nc = e_nc + beb_row_ref[...]
    s1_t_ref[0] = jax.lax.dot_general(e_nc, a1, (((1,), (1,)), ((), ())),
                                      preferred_element_type=jnp.float32)
    e_cn = jnp.dot(wemb, xb_cn, preferred_element_type=jnp.float32)
    e_cn = e_cn + beb_col_ref[...]
    s2 = jax.lax.dot_general(a2, e_cn, (((1,), (0,)), ((), ())),
                             preferred_element_type=jnp.float32)
    s2_n_ref[0] = s2 + battn_ref[0, 0]


def _main_body(xn_nc_ref, xn_cn_ref, x_nc_ref, sq_n_ref, sq_t_ref,
               s1_t_ref, s2_n_ref, wct_ref, bconv_ref,
               y_ref, sums_ref, sumsq_ref, d_scr, *, blk_r, n_nodes):
    b = pl.program_id(0)
    j = pl.program_id(1)
    C = xn_nc_ref.shape[2]

    xr = xn_nc_ref[0]
    xc = xn_cn_ref[0]
    d = sq_t_ref[0] + sq_n_ref[0] - 2.0 * jnp.dot(
        xr, xc, preferred_element_type=jnp.float32)
    d_scr[...] = d

    def step(_, m):
        dv = d_scr[...]
        return jnp.min(jnp.where(dv > m, dv, jnp.inf), axis=1, keepdims=True)

    t = jnp.min(d_scr[...], axis=1, keepdims=True)
    t = jax.lax.fori_loop(0, KNN - 1, step, t)

    d = d_scr[...]
    member = d <= t

    logit = s1_t_ref[0] + s2_n_ref[0]
    logit = jnp.where(logit >= 0, logit, 0.1 * logit)
    neg = jnp.float32(-1e30)
    mx = jnp.max(jnp.where(member, logit, neg), axis=1, keepdims=True)
    p = jnp.where(member, jnp.exp(logit - mx), 0.0)
    w = p / jnp.sum(p, axis=1, keepdims=True)

    x_full = x_nc_ref[0]
    agg = jnp.dot(w, x_full, preferred_element_type=jnp.float32)
    x_rows = x_nc_ref[0, pl.ds(j * blk_r, blk_r), :]

    wct = wct_ref[...]
    y = (jnp.dot(x_rows, wct[:C], preferred_element_type=jnp.float32)
         + jnp.dot(agg, wct[C:], preferred_element_type=jnp.float32)
         + bconv_ref[...])
    y_ref[0] = y

    @pl.when(jnp.logical_and(b == 0, j == 0))
    def _():
        sums_ref[...] = jnp.zeros_like(sums_ref)
        sumsq_ref[...] = jnp.zeros_like(sumsq_ref)

    sums_ref[...] += jnp.sum(y, axis=0, keepdims=True)
    sumsq_ref[...] += jnp.sum(y * y, axis=0, keepdims=True)


def _final_body(y_ref, sums_ref, sumsq_ref, gamma_ref, beta_ref, x_nc_ref,
                out_ref, *, count):
    mean = sums_ref[...] / count
    var = sumsq_ref[...] / count - mean * mean
    inv = jax.lax.rsqrt(var + 1e-5)
    y = y_ref[0]
    z = gamma_ref[...] * (y - mean) * inv + beta_ref[...]
    z = jnp.maximum(z, 0.0)
    out_ref[0] = z + x_nc_ref[0]


def kernel(x, W_emb, b_emb, W_attn, b_attn, W_conv, b_conv, gamma, beta):
    B, C, H, W = x.shape
    N = H * W
    x_cn = x.reshape(B, C, N)
    x_nc = x_cn.transpose(0, 2, 1)

    f32 = jnp.float32
    prep_out = pl.pallas_call(
        _prep_body,
        grid=(B,),
        in_specs=[
            pl.BlockSpec((1, C, N), lambda b: (b, 0, 0)),
            pl.BlockSpec((1, N, C), lambda b: (b, 0, 0)),
            pl.BlockSpec((C, C), lambda b: (0, 0)),
            pl.BlockSpec((1, 2 * C), lambda b: (0, 0)),
            pl.BlockSpec((C, 1), lambda b: (0, 0)),
            pl.BlockSpec((1, C), lambda b: (0, 0)),
            pl.BlockSpec((1, 1), lambda b: (0, 0)),
        ],
        out_specs=[
            pl.BlockSpec((1, C, N), lambda b: (b, 0, 0)),
            pl.BlockSpec((1, N, C), lambda b: (b, 0, 0)),
            pl.BlockSpec((1, 1, N), lambda b: (b, 0, 0)),
            pl.BlockSpec((1, N, 1), lambda b: (b, 0, 0)),
            pl.BlockSpec((1, N, 1), lambda b: (b, 0, 0)),
            pl.BlockSpec((1, 1, N), lambda b: (b, 0, 0)),
        ],
        out_shape=[
            jax.ShapeDtypeStruct((B, C, N), f32),
            jax.ShapeDtypeStruct((B, N, C), f32),
            jax.ShapeDtypeStruct((B, 1, N), f32),
            jax.ShapeDtypeStruct((B, N, 1), f32),
            jax.ShapeDtypeStruct((B, N, 1), f32),
            jax.ShapeDtypeStruct((B, 1, N), f32),
        ],
    )(x_cn, x_nc, W_emb, W_attn, b_emb.reshape(C, 1), b_emb.reshape(1, C),
      b_attn.reshape(1, 1))
    xn_cn, xn_nc, sq_n, sq_t, s1_t, s2_n = prep_out

    blk_r = 448
    nb = N // blk_r
    y_nc, sums, sumsq = pl.pallas_call(
        functools.partial(_main_body, blk_r=blk_r, n_nodes=N),
        grid=(B, nb),
        in_specs=[
            pl.BlockSpec((1, blk_r, C), lambda b, j: (b, j, 0)),
            pl.BlockSpec((1, C, N), lambda b, j: (b, 0, 0)),
            pl.BlockSpec((1, N, C), lambda b, j: (b, 0, 0)),
            pl.BlockSpec((1, 1, N), lambda b, j: (b, 0, 0)),
            pl.BlockSpec((1, blk_r, 1), lambda b, j: (b, j, 0)),
            pl.BlockSpec((1, blk_r, 1), lambda b, j: (b, j, 0)),
            pl.BlockSpec((1, 1, N), lambda b, j: (b, 0, 0)),
            pl.BlockSpec((2 * C, C), lambda b, j: (0, 0)),
            pl.BlockSpec((1, C), lambda b, j: (0, 0)),
        ],
        out_specs=[
            pl.BlockSpec((1, blk_r, C), lambda b, j: (b, j, 0)),
            pl.BlockSpec((1, C), lambda b, j: (0, 0)),
            pl.BlockSpec((1, C), lambda b, j: (0, 0)),
        ],
        out_shape=[
            jax.ShapeDtypeStruct((B, N, C), f32),
            jax.ShapeDtypeStruct((1, C), f32),
            jax.ShapeDtypeStruct((1, C), f32),
        ],
        scratch_shapes=[pltpu.VMEM((blk_r, N), f32)],
    )(xn_nc, xn_cn, x_nc, sq_n, sq_t, s1_t, s2_n, W_conv.T,
      b_conv.reshape(1, C))

    out_nc = pl.pallas_call(
        functools.partial(_final_body, count=float(B * N)),
        grid=(B,),
        in_specs=[
            pl.BlockSpec((1, N, C), lambda b: (b, 0, 0)),
            pl.BlockSpec((1, C), lambda b: (0, 0)),
            pl.BlockSpec((1, C), lambda b: (0, 0)),
            pl.BlockSpec((1, C), lambda b: (0, 0)),
            pl.BlockSpec((1, C), lambda b: (0, 0)),
            pl.BlockSpec((1, N, C), lambda b: (b, 0, 0)),
        ],
        out_specs=pl.BlockSpec((1, N, C), lambda b: (b, 0, 0)),
        out_shape=jax.ShapeDtypeStruct((B, N, C), f32),
    )(y_nc, sums, sumsq, gamma.reshape(1, C), beta.reshape(1, C), x_nc)

    return out_nc.transpose(0, 2, 1).reshape(B, C, H, W)
